# Initial kernel scaffold; baseline (speedup 1.0000x reference)
import jax
import jax.numpy as jnp
from jax.experimental import pallas as pl


def kernel(y_pred):
    raise NotImplementedError("write your pallas kernel here")


# speedup vs baseline: 13.3685x; 13.3685x over previous
import jax
import jax.numpy as jnp
from jax.experimental import pallas as pl
from jax.experimental.pallas import tpu as pltpu

B = 2
N = 5000
NPAD = 5120
C = 20
K = 200
KPAD = 256
CONF_T = 0.01
IOU_T = 0.45
NEG = -1e9
BIGI = 2**30
IMG = 512.0


def _body(yt_ref, cls_o, sel_o, x0_o, y0_o, x1_o, y1_o,
          S, X0, Y0, X1, Y1, A2, seff,
          cls_b, sel_b, bx0_b, by0_b, bx1_b, by1_b):
    lane = jax.lax.broadcasted_iota(jnp.int32, (1, 1, NPAD), 2)
    col = jax.lax.broadcasted_iota(jnp.int32, (1, 1, KPAD), 2)
    clsrow = jax.lax.broadcasted_iota(jnp.int32, (1, C, KPAD), 1)
    clsval = (clsrow + 1).astype(jnp.float32)

    cxo = yt_ref[21]
    cyo = yt_ref[22]
    wo = yt_ref[23]
    ho = yt_ref[24]
    acx = yt_ref[25]
    acy = yt_ref[26]
    aw = yt_ref[27]
    ah = yt_ref[28]
    vx = yt_ref[29]
    vy = yt_ref[30]
    vw = yt_ref[31]
    vh = yt_ref[32]
    cx = cxo * vx * aw + acx
    cy = cyo * vy * ah + acy
    w = jnp.exp(wo * vw) * aw
    h = jnp.exp(ho * vh) * ah
    x0 = (cx - 0.5 * w) * IMG
    y0 = (cy - 0.5 * h) * IMG
    x1 = (cx + 0.5 * w) * IMG
    y1 = (cy + 0.5 * h) * IMG
    X0[:, 0, :] = x0
    Y0[:, 0, :] = y0
    X1[:, 0, :] = x1
    Y1[:, 0, :] = y1
    A2[:, 0, :] = jnp.maximum(x1 - x0, 0.0) * jnp.maximum(y1 - y0, 0.0)

    for c in range(C):
        sc = yt_ref[1 + c]
        S[:, c, :] = jnp.where(sc > CONF_T, sc, NEG)

    z = jnp.zeros((B, C, KPAD), jnp.float32)
    cls_b[...] = z
    sel_b[...] = z
    bx0_b[...] = z
    by0_b[...] = z
    bx1_b[...] = z
    by1_b[...] = z

    def step(t, _):
        s = S[...]
        m = jnp.max(s, axis=2, keepdims=True)
        keep = m > CONF_T
        eq = s == m
        bidx = jnp.min(jnp.where(eq, lane, BIGI), axis=2, keepdims=True)
        oh = lane == bidx
        x0a = X0[...]
        y0a = Y0[...]
        x1a = X1[...]
        y1a = Y1[...]
        bx0 = jnp.sum(jnp.where(oh, x0a, 0.0), axis=2, keepdims=True)
        by0 = jnp.sum(jnp.where(oh, y0a, 0.0), axis=2, keepdims=True)
        bx1 = jnp.sum(jnp.where(oh, x1a, 0.0), axis=2, keepdims=True)
        by1 = jnp.sum(jnp.where(oh, y1a, 0.0), axis=2, keepdims=True)
        ixmin = jnp.maximum(bx0, x0a)
        iymin = jnp.maximum(by0, y0a)
        ixmax = jnp.minimum(bx1, x1a)
        iymax = jnp.minimum(by1, y1a)
        iw = jnp.maximum(ixmax - ixmin, 0.0)
        ih = jnp.maximum(iymax - iymin, 0.0)
        inter = iw * ih
        a1 = jnp.maximum(bx1 - bx0, 0.0) * jnp.maximum(by1 - by0, 0.0)
        union = a1 + A2[...] - inter
        iou = inter / jnp.maximum(union, 1e-8)
        ns = jnp.where(iou >= IOU_T, NEG, s)
        ns = jnp.where(oh, NEG, ns)
        S[...] = jnp.where(keep, ns, s)
        cm = (col == t) & keep
        sel_b[...] += jnp.where(cm, m, 0.0)
        cls_b[...] += jnp.where(cm, clsval, 0.0)
        bx0_b[...] += jnp.where(cm, bx0, 0.0)
        by0_b[...] += jnp.where(cm, by0, 0.0)
        bx1_b[...] += jnp.where(cm, bx1, 0.0)
        by1_b[...] += jnp.where(cm, by1, 0.0)
        return 0

    jax.lax.fori_loop(0, K, step, 0)

    flat = jnp.where(col < K, clsrow * K + col, BIGI)
    seff[...] = jnp.where(col < K, sel_b[...], NEG)
    zo = jnp.zeros((B, 1, KPAD), jnp.float32)
    cls_o[...] = zo
    sel_o[...] = zo
    x0_o[...] = zo
    y0_o[...] = zo
    x1_o[...] = zo
    y1_o[...] = zo

    def mstep(t, _):
        se = seff[...]
        mm = jnp.max(jnp.max(se, axis=2, keepdims=True), axis=1, keepdims=True)
        eqm = se == mm
        bfl = jnp.min(jnp.min(jnp.where(eqm, flat, BIGI), axis=2, keepdims=True),
                      axis=1, keepdims=True)
        ohm = flat == bfl
        seff[...] = jnp.where(ohm, NEG, se)

        def pick(buf):
            return jnp.sum(jnp.sum(jnp.where(ohm, buf, 0.0), axis=2, keepdims=True),
                           axis=1, keepdims=True)

        cmo = col == t
        cls_o[...] += jnp.where(cmo, pick(cls_b[...]), 0.0)
        sel_o[...] += jnp.where(cmo, pick(sel_b[...]), 0.0)
        x0_o[...] += jnp.where(cmo, pick(bx0_b[...]), 0.0)
        y0_o[...] += jnp.where(cmo, pick(by0_b[...]), 0.0)
        x1_o[...] += jnp.where(cmo, pick(bx1_b[...]), 0.0)
        y1_o[...] += jnp.where(cmo, pick(by1_b[...]), 0.0)
        return 0

    jax.lax.fori_loop(0, K, mstep, 0)


def _run(yt, interpret=False):
    f32 = jnp.float32
    outs = pl.pallas_call(
        _body,
        out_shape=[jax.ShapeDtypeStruct((B, 1, KPAD), f32) for _ in range(6)],
        scratch_shapes=[
            pltpu.VMEM((B, C, NPAD), f32),
            pltpu.VMEM((B, 1, NPAD), f32),
            pltpu.VMEM((B, 1, NPAD), f32),
            pltpu.VMEM((B, 1, NPAD), f32),
            pltpu.VMEM((B, 1, NPAD), f32),
            pltpu.VMEM((B, 1, NPAD), f32),
            pltpu.VMEM((B, C, KPAD), f32),
            pltpu.VMEM((B, C, KPAD), f32),
            pltpu.VMEM((B, C, KPAD), f32),
            pltpu.VMEM((B, C, KPAD), f32),
            pltpu.VMEM((B, C, KPAD), f32),
            pltpu.VMEM((B, C, KPAD), f32),
            pltpu.VMEM((B, C, KPAD), f32),
        ],
        interpret=interpret,
    )(yt)
    cls, sel, x0, y0, x1, y1 = outs
    det = jnp.stack([cls[:, 0, :], sel[:, 0, :], x0[:, 0, :],
                     y0[:, 0, :], x1[:, 0, :], y1[:, 0, :]], axis=-1)
    return det[:, :K, :]


def kernel(y_pred):
    yt = jnp.transpose(y_pred, (2, 0, 1))
    yt = jnp.pad(yt, ((0, 0), (0, 0), (0, NPAD - N)))
    return _run(yt)


# speedup vs baseline: 16.4527x; 1.2307x over previous
import jax
import jax.numpy as jnp
from jax.experimental import pallas as pl
from jax.experimental.pallas import tpu as pltpu

B = 2
N = 5000
NPAD = 5120
C = 20
K = 200
KPAD = 256
CONF_T = 0.01
IOU_T = 0.45
NEG = -1e9
BIGI = 2**30
IMG = 512.0


def _body(yt_ref, cls_o, sel_o, x0_o, y0_o, x1_o, y1_o,
          S, X0, Y0, X1, Y1, A2, seff,
          cls_b, sel_b, bx0_b, by0_b, bx1_b, by1_b):
    lane = jax.lax.broadcasted_iota(jnp.int32, (1, 1, NPAD), 2)
    col = jax.lax.broadcasted_iota(jnp.int32, (1, 1, KPAD), 2)
    clsrow = jax.lax.broadcasted_iota(jnp.int32, (1, C, KPAD), 1)
    clsval = (clsrow + 1).astype(jnp.float32)

    cxo = yt_ref[21]
    cyo = yt_ref[22]
    wo = yt_ref[23]
    ho = yt_ref[24]
    acx = yt_ref[25]
    acy = yt_ref[26]
    aw = yt_ref[27]
    ah = yt_ref[28]
    vx = yt_ref[29]
    vy = yt_ref[30]
    vw = yt_ref[31]
    vh = yt_ref[32]
    cx = cxo * vx * aw + acx
    cy = cyo * vy * ah + acy
    w = jnp.exp(wo * vw) * aw
    h = jnp.exp(ho * vh) * ah
    x0 = (cx - 0.5 * w) * IMG
    y0 = (cy - 0.5 * h) * IMG
    x1 = (cx + 0.5 * w) * IMG
    y1 = (cy + 0.5 * h) * IMG
    X0[:, 0, :] = x0
    Y0[:, 0, :] = y0
    X1[:, 0, :] = x1
    Y1[:, 0, :] = y1
    A2[:, 0, :] = jnp.maximum(x1 - x0, 0.0) * jnp.maximum(y1 - y0, 0.0)

    for c in range(C):
        sc = yt_ref[1 + c]
        S[:, c, :] = jnp.where(sc > CONF_T, sc, NEG)

    z = jnp.zeros((B, C, KPAD), jnp.float32)
    cls_b[...] = z
    sel_b[...] = z
    bx0_b[...] = z
    by0_b[...] = z
    bx1_b[...] = z
    by1_b[...] = z

    def step(t, _):
        s = S[...]
        m = jnp.max(s, axis=2, keepdims=True)
        keep = m > CONF_T
        eq = s == m
        bidx = jnp.min(jnp.where(eq, lane, BIGI), axis=2, keepdims=True)
        oh = lane == bidx
        x0a = X0[...]
        y0a = Y0[...]
        x1a = X1[...]
        y1a = Y1[...]
        bx0 = jnp.sum(jnp.where(oh, x0a, 0.0), axis=2, keepdims=True)
        by0 = jnp.sum(jnp.where(oh, y0a, 0.0), axis=2, keepdims=True)
        bx1 = jnp.sum(jnp.where(oh, x1a, 0.0), axis=2, keepdims=True)
        by1 = jnp.sum(jnp.where(oh, y1a, 0.0), axis=2, keepdims=True)
        ixmin = jnp.maximum(bx0, x0a)
        iymin = jnp.maximum(by0, y0a)
        ixmax = jnp.minimum(bx1, x1a)
        iymax = jnp.minimum(by1, y1a)
        iw = jnp.maximum(ixmax - ixmin, 0.0)
        ih = jnp.maximum(iymax - iymin, 0.0)
        inter = iw * ih
        a1 = jnp.maximum(bx1 - bx0, 0.0) * jnp.maximum(by1 - by0, 0.0)
        union = a1 + A2[...] - inter
        iou = inter / jnp.maximum(union, 1e-8)
        ns = jnp.where(iou >= IOU_T, NEG, s)
        ns = jnp.where(oh, NEG, ns)
        S[...] = jnp.where(keep, ns, s)
        cm = (col == t) & keep
        sel_b[...] += jnp.where(cm, m, 0.0)
        cls_b[...] += jnp.where(cm, clsval, 0.0)
        bx0_b[...] += jnp.where(cm, bx0, 0.0)
        by0_b[...] += jnp.where(cm, by0, 0.0)
        bx1_b[...] += jnp.where(cm, bx1, 0.0)
        by1_b[...] += jnp.where(cm, by1, 0.0)
        return 0

    jax.lax.fori_loop(0, K, step, 0)

    flat = jnp.where(col < K, clsrow * K + col, BIGI)
    seff[...] = jnp.where(col < K, sel_b[...], NEG)
    zo = jnp.zeros((B, 1, KPAD), jnp.float32)
    cls_o[...] = zo
    sel_o[...] = zo
    x0_o[...] = zo
    y0_o[...] = zo
    x1_o[...] = zo
    y1_o[...] = zo

    def mstep(t, _):
        se = seff[...]
        mm = jnp.max(jnp.max(se, axis=2, keepdims=True), axis=1, keepdims=True)
        eqm = se == mm
        bfl = jnp.min(jnp.min(jnp.where(eqm, flat, BIGI), axis=2, keepdims=True),
                      axis=1, keepdims=True)
        ohm = flat == bfl
        seff[...] = jnp.where(ohm, NEG, se)

        def pick(buf):
            return jnp.sum(jnp.sum(jnp.where(ohm, buf, 0.0), axis=2, keepdims=True),
                           axis=1, keepdims=True)

        cmo = col == t
        cls_o[...] += jnp.where(cmo, pick(cls_b[...]), 0.0)
        sel_o[...] += jnp.where(cmo, pick(sel_b[...]), 0.0)
        x0_o[...] += jnp.where(cmo, pick(bx0_b[...]), 0.0)
        y0_o[...] += jnp.where(cmo, pick(by0_b[...]), 0.0)
        x1_o[...] += jnp.where(cmo, pick(bx1_b[...]), 0.0)
        y1_o[...] += jnp.where(cmo, pick(by1_b[...]), 0.0)
        return 0

    jax.lax.fori_loop(0, K, mstep, 0)


def _run(yt, interpret=False):
    f32 = jnp.float32
    outs = pl.pallas_call(
        _body,
        out_shape=[jax.ShapeDtypeStruct((B, 1, KPAD), f32) for _ in range(6)],
        scratch_shapes=[
            pltpu.VMEM((B, C, NPAD), f32),
            pltpu.VMEM((B, 1, NPAD), f32),
            pltpu.VMEM((B, 1, NPAD), f32),
            pltpu.VMEM((B, 1, NPAD), f32),
            pltpu.VMEM((B, 1, NPAD), f32),
            pltpu.VMEM((B, 1, NPAD), f32),
            pltpu.VMEM((B, C, KPAD), f32),
            pltpu.VMEM((B, C, KPAD), f32),
            pltpu.VMEM((B, C, KPAD), f32),
            pltpu.VMEM((B, C, KPAD), f32),
            pltpu.VMEM((B, C, KPAD), f32),
            pltpu.VMEM((B, C, KPAD), f32),
            pltpu.VMEM((B, C, KPAD), f32),
        ],
        interpret=interpret,
    )(yt)
    cls, sel, x0, y0, x1, y1 = outs
    det = jnp.stack([cls[:, 0, :], sel[:, 0, :], x0[:, 0, :],
                     y0[:, 0, :], x1[:, 0, :], y1[:, 0, :]], axis=-1)
    return det[:, :K, :]


def _kernel_tc(y_pred):
    yt = jnp.transpose(y_pred, (2, 0, 1))
    yt = jnp.pad(yt, ((0, 0), (0, 0), (0, NPAD - N)))
    return _run(yt)


from sc_kernel import kernel_sc as _kernel_sc

kernel = _kernel_sc
